# baseline (speedup 1.0000x reference)
import functools
import math

import numpy as np

import jax
import jax.numpy as jnp
from jax import lax
from jax.experimental import pallas as pl
from jax.experimental.pallas import tpu as pltpu
from jax.experimental.pallas import tpu_sc as plsc

_N = 2048
_H = 16
_BW = 4096
_GW = 2176
_XW = 4231
_DEPTH = 16


def _bucket_index(relative_position, num_buckets=32, max_distance=128):
    ret = 0
    nneg = -relative_position
    num_buckets //= 2
    ret += (nneg < 0).astype(np.int32) * num_buckets
    nn = np.abs(nneg)
    max_exact = num_buckets // 2
    is_small = nn < max_exact
    with np.errstate(divide="ignore", invalid="ignore"):
        val_if_large = max_exact + (
            np.log(nn.astype(np.float32) / max_exact)
            / math.log(max_distance / max_exact)
            * (num_buckets - max_exact)
        ).astype(np.int32)
    val_if_large = np.minimum(val_if_large, np.full_like(val_if_large, num_buckets - 1))
    ret += np.where(is_small, nn, val_if_large)
    return ret


def _static_bucket_rows():
    x = np.arange(_XW + 7, dtype=np.int32)
    rel = np.maximum((_N - 1) - x, -(_N - 1))
    widx = _bucket_index(rel)
    return np.stack([widx[7 - r:7 - r + _XW] for r in range(8)])


_BIDX_ROWS = _static_bucket_rows()


@functools.partial(
    pl.kernel,
    mesh=plsc.VectorSubcoreMesh(core_axis_name="c", subcore_axis_name="s"),
    out_type=jax.ShapeDtypeStruct((_H, 16, _BW // 16, 128), jnp.float32),
    compiler_params=pltpu.CompilerParams(
        needs_layout_passes=False, use_tc_tiling_on_sc=False),
    scratch_types=[
        pltpu.VMEM((32 * _H,), jnp.float32),
        pltpu.VMEM((8, _GW), jnp.int32),
        pltpu.VMEM((8, _GW), jnp.float32),
        pltpu.SemaphoreType.DMA,
    ],
)
def _gather_sc_kernel(table_hbm, bidx_hbm, b_hbm, table_v, bidx_v, revr_v, sem):
    c = lax.axis_index("c")
    s = lax.axis_index("s")
    h = s
    h32 = h * 32
    g0 = c * 2048

    pltpu.sync_copy(table_hbm, table_v)
    pltpu.sync_copy(bidx_hbm.at[:, pl.ds(g0, _GW)], bidx_v)

    def gather_body(k, carry):
        base = k * 16
        for r in range(8):
            idx16 = bidx_v[r, pl.ds(base, 16)]
            vals = plsc.load_gather(table_v, [idx16 + h32])
            revr_v[r, pl.ds(base, 16)] = vals
        return carry

    lax.fori_loop(0, _GW // 16, gather_body, 0)

    def fire(si, t):
        q = pl.multiple_of(120 - 8 * si + 128 * t, 8)
        pltpu.async_copy(
            revr_v.at[:, pl.ds(q, 128)],
            b_hbm.at[h, si, pl.ds(8 * (c * 16 + t), 8), :],
            sem,
        )

    def drain_one():
        pltpu.make_async_copy(
            b_hbm.at[0, 0, pl.ds(0, 8), :], revr_v.at[:, pl.ds(0, 128)], sem
        ).wait()

    for t in range(2):
        for si in range(16):
            fire(si, t)

    def dma_body(t, carry):
        for _ in range(16):
            drain_one()
        for si in range(16):
            fire(si, t + 2)
        return carry

    lax.fori_loop(0, 16 - 2, dma_body, 0)
    for _ in range(32):
        drain_one()


def _materialize_tc_body(b_ref, out_ref):
    def body(k, carry):
        s_ = k % 16
        m = k // 16
        tj0 = 15 - m
        for tj in range(16):
            out_ref[0, pl.ds(8 * k, 8), 128 * tj:128 * (tj + 1)] = (
                b_ref[0, s_, pl.ds(8 * (tj0 + tj), 8), :]
            )
        return carry

    lax.fori_loop(0, _N // 8, body, 0)


def kernel(n, relative_attention_bias):
    del n
    table = relative_attention_bias.astype(jnp.float32)
    bidx = jnp.asarray(_BIDX_ROWS, dtype=jnp.int32)

    b = _gather_sc_kernel(table.T.reshape(-1), bidx)

    return pl.pallas_call(
        _materialize_tc_body,
        out_shape=jax.ShapeDtypeStruct((_H, _N, _N), jnp.float32),
        grid=(_H,),
        in_specs=[pl.BlockSpec(
            (1, 16, _BW // 16, 128), lambda hh: (hh, 0, 0, 0))],
        out_specs=pl.BlockSpec((1, _N, _N), lambda hh: (hh, 0, 0)),
        compiler_params=pltpu.CompilerParams(
            dimension_semantics=("arbitrary",)),
    )(b)


# speedup vs baseline: 1.0053x; 1.0053x over previous
import functools
import math

import numpy as np

import jax
import jax.numpy as jnp
from jax import lax
from jax.experimental import pallas as pl
from jax.experimental.pallas import tpu as pltpu
from jax.experimental.pallas import tpu_sc as plsc

_N = 2048
_H = 16
_BW = 4096
_GW = 2176
_XW = 4231
_DEPTH = 16


def _bucket_index(relative_position, num_buckets=32, max_distance=128):
    ret = 0
    nneg = -relative_position
    num_buckets //= 2
    ret += (nneg < 0).astype(np.int32) * num_buckets
    nn = np.abs(nneg)
    max_exact = num_buckets // 2
    is_small = nn < max_exact
    with np.errstate(divide="ignore", invalid="ignore"):
        val_if_large = max_exact + (
            np.log(nn.astype(np.float32) / max_exact)
            / math.log(max_distance / max_exact)
            * (num_buckets - max_exact)
        ).astype(np.int32)
    val_if_large = np.minimum(val_if_large, np.full_like(val_if_large, num_buckets - 1))
    ret += np.where(is_small, nn, val_if_large)
    return ret


def _static_bucket_rows():
    x = np.arange(_XW + 7, dtype=np.int32)
    rel = np.maximum((_N - 1) - x, -(_N - 1))
    widx = _bucket_index(rel)
    return np.stack([widx[7 - r:7 - r + _XW] for r in range(8)])


_BIDX_ROWS = _static_bucket_rows()


@functools.partial(
    pl.kernel,
    mesh=plsc.VectorSubcoreMesh(core_axis_name="c", subcore_axis_name="s"),
    out_type=jax.ShapeDtypeStruct((_H, 16, _BW // 16, 128), jnp.float32),
    compiler_params=pltpu.CompilerParams(
        needs_layout_passes=False, use_tc_tiling_on_sc=False),
    scratch_types=[
        pltpu.VMEM((32 * _H,), jnp.float32),
        pltpu.VMEM((8, _GW), jnp.int32),
        pltpu.VMEM((8, _GW), jnp.float32),
        pltpu.SemaphoreType.DMA,
    ],
)
def _gather_sc_kernel(table_hbm, bidx_hbm, b_hbm, table_v, bidx_v, revr_v, sem):
    c = lax.axis_index("c")
    s = lax.axis_index("s")
    h = s
    h32 = h * 32
    g0 = c * 2048

    pltpu.sync_copy(table_hbm, table_v)
    pltpu.sync_copy(bidx_hbm.at[:, pl.ds(g0, _GW)], bidx_v)

    def gather_body(k, carry):
        base = k * 16
        for r in range(8):
            idx16 = bidx_v[r, pl.ds(base, 16)]
            vals = plsc.load_gather(table_v, [idx16 + h32])
            revr_v[r, pl.ds(base, 16)] = vals
        return carry

    lax.fori_loop(0, _GW // 16, gather_body, 0)

    def fire(si, t):
        q = pl.multiple_of(120 - 8 * si + 128 * t, 8)
        pltpu.async_copy(
            revr_v.at[:, pl.ds(q, 128)],
            b_hbm.at[h, si, pl.ds(8 * (c * 16 + t), 8), :],
            sem,
        )

    def drain_one():
        pltpu.make_async_copy(
            b_hbm.at[0, 0, pl.ds(0, 8), :], revr_v.at[:, pl.ds(0, 128)], sem
        ).wait()

    for t in range(1):
        for si in range(16):
            fire(si, t)

    def dma_body(t, carry):
        for _ in range(16):
            drain_one()
        for si in range(16):
            fire(si, t + 1)
        return carry

    lax.fori_loop(0, 16 - 1, dma_body, 0)
    for _ in range(16):
        drain_one()


def _materialize_tc_body(b_ref, out_ref):
    def body(k, carry):
        s_ = k % 16
        m = k // 16
        tj0 = 15 - m
        for tj in range(16):
            out_ref[0, pl.ds(8 * k, 8), 128 * tj:128 * (tj + 1)] = (
                b_ref[0, s_, pl.ds(8 * (tj0 + tj), 8), :]
            )
        return carry

    lax.fori_loop(0, _N // 8, body, 0)


def kernel(n, relative_attention_bias):
    del n
    table = relative_attention_bias.astype(jnp.float32)
    bidx = jnp.asarray(_BIDX_ROWS, dtype=jnp.int32)

    b = _gather_sc_kernel(table.T.reshape(-1), bidx)

    return pl.pallas_call(
        _materialize_tc_body,
        out_shape=jax.ShapeDtypeStruct((_H, _N, _N), jnp.float32),
        grid=(_H,),
        in_specs=[pl.BlockSpec(
            (1, 16, _BW // 16, 128), lambda hh: (hh, 0, 0, 0))],
        out_specs=pl.BlockSpec((1, _N, _N), lambda hh: (hh, 0, 0)),
        compiler_params=pltpu.CompilerParams(
            dimension_semantics=("arbitrary",)),
    )(b)
